# baseline (speedup 1.0000x reference)
import functools

import jax
import jax.numpy as jnp
from jax import lax
from jax.experimental import pallas as pl
from jax.experimental.pallas import tpu as pltpu
from jax.experimental.pallas import tpu_sc as plsc

_CONF = 0.9


def _sc_body(B, V, C, R, NC, out_hbm, t_hbm, oh_hbm,
             s_out, g_out, oht_out, i_out,
             buf0, buf1, t_v, s_v, g_v, oht_v, i_v, sem0, sem1, semg):
    wid = lax.axis_index("s") * NC + lax.axis_index("c")
    base = wid * R
    ngrp = R // 16
    vc = (V // 128) * 128
    nfull = vc // C
    chunks = [(c * C, C, C) for c in range(nfull)]
    if vc > nfull * C:
        rem = vc - nfull * C
        chunks.append((nfull * C, rem, rem))
    nch = len(chunks)
    ign = V - 100
    ign_c = next(i for i, (st, _, va) in enumerate(chunks)
                 if st <= ign < st + va)
    ign_off = ign - chunks[ign_c][0]

    pltpu.sync_copy(t_hbm.at[pl.ds(base, R)], t_v)

    bufs = (buf0, buf1)
    sems = (sem0, sem1)

    def start(q):
        grp, c = q // nch, q % nch
        st, sz, _ = chunks[c]
        dst = bufs[q % 2]
        if sz != C:
            dst = dst.at[:, pl.ds(0, sz)]
        return pltpu.async_copy(
            out_hbm.at[pl.ds(base + grp * 16, 16), pl.ds(st, sz)],
            dst, sems[q % 2])

    lanes = lax.iota(jnp.int32, 16)

    handles = {0: start(0)}
    nq = ngrp * nch
    for q in range(nq):
        grp, c = q // nch, q % nch
        if q + 1 < nq:
            handles[q + 1] = start(q + 1)
        handles[q].wait()
        buf = bufs[q % 2]

        if c == 0:
            accs = [jnp.zeros((16,), jnp.float32) for _ in range(16)]

        st, _, va = chunks[c]

        def red_body(j, a):
            return tuple(
                a[r] + buf[r, pl.ds(j * 16, 16)] for r in range(16))
        accs = list(lax.fori_loop(0, va // 16, red_body, tuple(accs)))

        if c == nch - 1:
            svec = jnp.zeros((16,), jnp.float32)
            for r in range(16):
                svec = jnp.where(lanes == r, jnp.sum(accs[r]), svec)
            s_v[pl.ds(grp * 16, 16)] = svec

    iwb = min((ign // 128) * 128, vc - 128)
    ijb, ilane = ((ign - iwb) // 16) * 16, (ign - iwb) % 16
    for grp in range(ngrp):
        tvec = t_v[pl.ds(grp * 16, 16)]
        rows = pl.ds(base + grp * 16, 16)
        wh = []
        for r in range(16):
            wb = pl.multiple_of(
                jnp.clip((tvec[r] // 128) * 128, 0, vc - 128), 128)
            wh.append(pltpu.async_copy(
                out_hbm.at[rows, pl.ds(wb, 128)],
                bufs[grp % 2].at[:, pl.ds(r * 128, 128)], sems[grp % 2]))
        hign = pltpu.async_copy(
            out_hbm.at[rows, pl.ds(iwb, 128)],
            bufs[grp % 2].at[:, pl.ds(16 * 128, 128)], semg)
        for h in wh:
            h.wait()
        hign.wait()

        gvec = jnp.zeros((16,), jnp.float32)
        ivec = jnp.zeros((16,), jnp.float32)
        for r in range(16):
            t_r = tvec[r]
            wb = jnp.clip((t_r // 128) * 128, 0, vc - 128)
            off = t_r - wb
            jb = jnp.clip((off // 16) * 16, 0, 112)
            v16 = bufs[grp % 2][r, pl.ds(pl.multiple_of(r * 128 + jb, 16), 16)]
            hit = jnp.sum(v16 * (lanes == off - jb).astype(jnp.float32))
            gvec = jnp.where(lanes == r, jnp.where(t_r < vc, hit, 0.0), gvec)
            iv16 = bufs[grp % 2][r, pl.ds(16 * 128 + ijb, 16)]
            ival = jnp.sum(iv16 * (lanes == ilane).astype(jnp.float32))
            ivec = jnp.where(lanes == r, ival, ivec)
        g_v[pl.ds(grp * 16, 16)] = gvec
        i_v[pl.ds(grp * 16, 16)] = ivec

    pltpu.async_copy(oh_hbm.at[t_v], oht_v, semg).wait()
    pltpu.sync_copy(s_v, s_out.at[pl.ds(base, R)])
    pltpu.sync_copy(g_v, g_out.at[pl.ds(base, R)])
    pltpu.sync_copy(oht_v, oht_out.at[pl.ds(base, R)])
    pltpu.sync_copy(i_v, i_out.at[pl.ds(base, R)])


def _combine_body(B, V, vc, oh_ref, s_ref, g_ref, oht_ref, i_ref,
                  tail_ref, t_ref, res_ref):
    oh = oh_ref[...]
    s = oh[0, 0]
    safe = jnp.where(oh > 0, oh, 1.0)
    kk = jnp.sum(jnp.where(oh > 0, oh * jnp.log(safe), 0.0))

    tail = tail_ref[...]
    tw = tail.shape[1]
    tcol = t_ref[...]
    cols = jax.lax.broadcasted_iota(jnp.int32, (B, tw), 1) + vc
    tmask = cols == tcol
    g_tail = jnp.sum(jnp.where(tmask, tail, 0.0), axis=1, keepdims=True)

    t_tot = (jnp.sum(s_ref[...])
             + jnp.sum(jnp.where(cols < V, tail, 0.0)))
    ci = jnp.sum(i_ref[...])
    w = s * (t_tot - ci)

    g = g_ref[...] + g_tail
    oht = oht_ref[...]
    safe_t = jnp.where(oht > 0, oht, 1.0)
    xlogy_t = jnp.where(oht > 0, oht * jnp.log(safe_t), 0.0)
    corr = _CONF * jnp.log(_CONF) - _CONF * g - xlogy_t + oht * g
    res_ref[0, 0] = B * kk - w + jnp.sum(corr)


def kernel(output, target, one_hot):
    B, V = output.shape
    info = plsc.get_sparse_core_info()
    NC, NS = info.num_cores, info.num_subcores
    NW = NC * NS
    R = B // NW
    C = 3584

    sc = functools.partial(
        pl.kernel,
        out_type=[jax.ShapeDtypeStruct((B,), jnp.float32)] * 4,
        mesh=plsc.VectorSubcoreMesh(core_axis_name="c", subcore_axis_name="s"),
        compiler_params=pltpu.CompilerParams(needs_layout_passes=False),
        scratch_types=[
            pltpu.VMEM((16, C), jnp.float32),
            pltpu.VMEM((16, C), jnp.float32),
            pltpu.VMEM((R,), jnp.int32),
            pltpu.VMEM((R,), jnp.float32),
            pltpu.VMEM((R,), jnp.float32),
            pltpu.VMEM((R,), jnp.float32),
            pltpu.VMEM((R,), jnp.float32),
            pltpu.SemaphoreType.DMA,
            pltpu.SemaphoreType.DMA,
            pltpu.SemaphoreType.DMA,
        ],
    )(functools.partial(_sc_body, B, V, C, R, NC))
    s_arr, g_arr, oht_arr, i_arr = sc(output, target, one_hot)

    vc = (V // 128) * 128
    tw = 128
    res = pl.pallas_call(
        functools.partial(_combine_body, B, V, vc),
        grid=(1,),
        in_specs=[
            pl.BlockSpec((1, V), lambda k: (0, 0)),
            pl.BlockSpec((1, B), lambda k: (0, 0)),
            pl.BlockSpec((B, 1), lambda k: (0, 0)),
            pl.BlockSpec((B, 1), lambda k: (0, 0)),
            pl.BlockSpec((1, B), lambda k: (0, 0)),
            pl.BlockSpec((B, tw), lambda k: (0, vc // tw)),
            pl.BlockSpec((B, 1), lambda k: (0, 0)),
        ],
        out_specs=pl.BlockSpec(memory_space=pltpu.SMEM),
        out_shape=jax.ShapeDtypeStruct((1, 1), jnp.float32),
    )(one_hot.reshape(1, V), s_arr.reshape(1, B), g_arr.reshape(B, 1),
      oht_arr.reshape(B, 1), i_arr.reshape(1, B), output,
      target.reshape(B, 1))
    return res[0, 0]


# speedup vs baseline: 1.0375x; 1.0375x over previous
import functools

import jax
import jax.numpy as jnp
from jax import lax
from jax.experimental import pallas as pl
from jax.experimental.pallas import tpu as pltpu
from jax.experimental.pallas import tpu_sc as plsc

_CONF = 0.9
_BS = 512


def _sc_body(B, V, C, R, NC, out_hbm, t_hbm, oh_hbm,
             s_out, g_out, oht_out, i_out,
             buf0, buf1, t_v, s_v, g_v, oht_v, i_v, sem0, sem1, semg):
    wid = lax.axis_index("s") * NC + lax.axis_index("c")
    base = wid * R
    ngrp = R // 16
    vc = (V // 128) * 128
    nfull = vc // C
    chunks = [(c * C, C, C) for c in range(nfull)]
    if vc > nfull * C:
        rem = vc - nfull * C
        chunks.append((nfull * C, rem, rem))
    nch = len(chunks)
    ign = V - 100

    pltpu.sync_copy(t_hbm.at[pl.ds(base, R)], t_v)

    bufs = (buf0, buf1)
    sems = (sem0, sem1)

    def start(q):
        grp, c = q // nch, q % nch
        st, sz, _ = chunks[c]
        dst = bufs[q % 2]
        if sz != C:
            dst = dst.at[:, pl.ds(0, sz)]
        return pltpu.async_copy(
            out_hbm.at[pl.ds(base + grp * 16, 16), pl.ds(st, sz)],
            dst, sems[q % 2])

    lanes = lax.iota(jnp.int32, 16)

    handles = {0: start(0)}
    nq = ngrp * nch
    for q in range(nq):
        grp, c = q // nch, q % nch
        if q + 1 < nq:
            handles[q + 1] = start(q + 1)
        handles[q].wait()
        buf = bufs[q % 2]

        if c == 0:
            accs = [jnp.zeros((16,), jnp.float32) for _ in range(16)]

        st, _, va = chunks[c]

        def red_body(j, a):
            return tuple(
                a[r] + buf[r, pl.ds(j * 16, 16)] for r in range(16))
        accs = list(lax.fori_loop(0, va // 16, red_body, tuple(accs)))

        if c == nch - 1:
            svec = jnp.zeros((16,), jnp.float32)
            for r in range(16):
                svec = jnp.where(lanes == r, jnp.sum(accs[r]), svec)
            s_v[pl.ds(grp * 16, 16)] = svec

    iwb = min((ign // 128) * 128, vc - 128)
    ijb, ilane = ((ign - iwb) // 16) * 16, (ign - iwb) % 16
    for grp in range(ngrp):
        tvec = t_v[pl.ds(grp * 16, 16)]
        rows = pl.ds(base + grp * 16, 16)
        wh = []
        for r in range(16):
            wb = pl.multiple_of(
                jnp.clip((tvec[r] // 128) * 128, 0, vc - 128), 128)
            wh.append(pltpu.async_copy(
                out_hbm.at[rows, pl.ds(wb, 128)],
                bufs[grp % 2].at[:, pl.ds(r * 128, 128)], sems[grp % 2]))
        hign = pltpu.async_copy(
            out_hbm.at[rows, pl.ds(iwb, 128)],
            bufs[grp % 2].at[:, pl.ds(16 * 128, 128)], semg)
        for h in wh:
            h.wait()
        hign.wait()

        gvec = jnp.zeros((16,), jnp.float32)
        ivec = jnp.zeros((16,), jnp.float32)
        for r in range(16):
            t_r = tvec[r]
            wb = jnp.clip((t_r // 128) * 128, 0, vc - 128)
            off = t_r - wb
            jb = jnp.clip((off // 16) * 16, 0, 112)
            v16 = bufs[grp % 2][r, pl.ds(pl.multiple_of(r * 128 + jb, 16), 16)]
            hit = jnp.sum(v16 * (lanes == off - jb).astype(jnp.float32))
            gvec = jnp.where(lanes == r, jnp.where(t_r < vc, hit, 0.0), gvec)
            iv16 = bufs[grp % 2][r, pl.ds(16 * 128 + ijb, 16)]
            ival = jnp.sum(iv16 * (lanes == ilane).astype(jnp.float32))
            ivec = jnp.where(lanes == r, ival, ivec)
        g_v[pl.ds(grp * 16, 16)] = gvec
        i_v[pl.ds(grp * 16, 16)] = ivec

    pltpu.async_copy(oh_hbm.at[t_v], oht_v, semg).wait()
    pltpu.sync_copy(s_v, s_out.at[pl.ds(base, R)])
    pltpu.sync_copy(g_v, g_out.at[pl.ds(base, R)])
    pltpu.sync_copy(oht_v, oht_out.at[pl.ds(base, R)])
    pltpu.sync_copy(i_v, i_out.at[pl.ds(base, R)])


def _tc_body(nblk, Br, V, Wb, out_ref, t_ref, oh_ref, res_ref,
             accw_ref, acck_ref, g_ref, oht_ref):
    k = pl.program_id(0)

    @pl.when(k == 0)
    def _init():
        accw_ref[0, 0] = 0.0
        acck_ref[0, 0] = 0.0
        g_ref[...] = jnp.zeros_like(g_ref)
        oht_ref[...] = jnp.zeros_like(oht_ref)

    x = out_ref[...]
    oh = oh_ref[...]
    col = jax.lax.broadcasted_iota(jnp.int32, (1, Wb), 1) + k * Wb
    valid = col < V

    colsum = jnp.sum(x, axis=0, keepdims=True)
    accw_ref[0, 0] += jnp.sum(jnp.where(valid, colsum * oh, 0.0))

    safe = jnp.where(oh > 0, oh, 1.0)
    acck_ref[0, 0] += jnp.sum(jnp.where(valid & (oh > 0),
                                        oh * jnp.log(safe), 0.0))

    tcol = t_ref[...]
    cols2 = jax.lax.broadcasted_iota(jnp.int32, (Br, Wb), 1) + k * Wb
    mask = cols2 == tcol
    g_ref[...] += jnp.sum(jnp.where(mask, x, 0.0), axis=1, keepdims=True)
    ohb = jnp.broadcast_to(oh, (Br, Wb))
    oht_ref[...] += jnp.sum(jnp.where(mask, ohb, 0.0), axis=1, keepdims=True)

    @pl.when(k == nblk - 1)
    def _fin():
        g = g_ref[...]
        oht = oht_ref[...]
        safe_t = jnp.where(oht > 0, oht, 1.0)
        xlogy_t = jnp.where(oht > 0, oht * jnp.log(safe_t), 0.0)
        corr = _CONF * jnp.log(_CONF) - _CONF * g - xlogy_t + oht * g
        res_ref[0, 0] = accw_ref[0, 0]
        res_ref[0, 1] = acck_ref[0, 0]
        res_ref[0, 2] = jnp.sum(corr)


def _combine_body(B, Bs, V, vc, tc_ref, oh_ref, s_ref, g_ref, oht_ref,
                  i_ref, tail_ref, t_ref, res_ref):
    w_tc, kk, corr_tc = tc_ref[0, 0], tc_ref[0, 1], tc_ref[0, 2]
    s = oh_ref[0, 0]

    tail = tail_ref[...]
    tw = tail.shape[1]
    tcol = t_ref[...]
    cols = jax.lax.broadcasted_iota(jnp.int32, (Bs, tw), 1) + vc
    tmask = cols == tcol
    g_tail = jnp.sum(jnp.where(tmask, tail, 0.0), axis=1, keepdims=True)

    t_sc = (jnp.sum(s_ref[...])
            + jnp.sum(jnp.where(cols < V, tail, 0.0)))
    ci = jnp.sum(i_ref[...])
    w_sc = s * (t_sc - ci)

    g = g_ref[...] + g_tail
    oht = oht_ref[...]
    safe_t = jnp.where(oht > 0, oht, 1.0)
    xlogy_t = jnp.where(oht > 0, oht * jnp.log(safe_t), 0.0)
    corr_sc = jnp.sum(_CONF * jnp.log(_CONF) - _CONF * g - xlogy_t + oht * g)
    res_ref[0, 0] = B * kk - (w_tc + w_sc) + corr_tc + corr_sc


def kernel(output, target, one_hot):
    B, V = output.shape
    Bs = _BS
    Br = B - Bs
    info = plsc.get_sparse_core_info()
    NC, NS = info.num_cores, info.num_subcores
    NW = NC * NS
    R = Bs // NW
    C = 3584

    sc = functools.partial(
        pl.kernel,
        out_type=[jax.ShapeDtypeStruct((Bs,), jnp.float32)] * 4,
        mesh=plsc.VectorSubcoreMesh(core_axis_name="c", subcore_axis_name="s"),
        compiler_params=pltpu.CompilerParams(needs_layout_passes=False),
        scratch_types=[
            pltpu.VMEM((16, C), jnp.float32),
            pltpu.VMEM((16, C), jnp.float32),
            pltpu.VMEM((R,), jnp.int32),
            pltpu.VMEM((R,), jnp.float32),
            pltpu.VMEM((R,), jnp.float32),
            pltpu.VMEM((R,), jnp.float32),
            pltpu.VMEM((R,), jnp.float32),
            pltpu.SemaphoreType.DMA,
            pltpu.SemaphoreType.DMA,
            pltpu.SemaphoreType.DMA,
        ],
    )(functools.partial(_sc_body, B, V, C, R, NC))
    s_arr, g_arr, oht_arr, i_arr = sc(output, target, one_hot)

    Wb = 4096
    nblk = pl.cdiv(V, Wb)
    oh2 = one_hot.reshape(1, V)
    tc_out = pl.pallas_call(
        functools.partial(_tc_body, nblk, Br, V, Wb),
        grid=(nblk,),
        in_specs=[
            pl.BlockSpec((Br, Wb), lambda k: (1, k)),
            pl.BlockSpec((Br, 1), lambda k: (1, 0)),
            pl.BlockSpec((1, Wb), lambda k: (0, k)),
        ],
        out_specs=pl.BlockSpec(memory_space=pltpu.SMEM),
        out_shape=jax.ShapeDtypeStruct((1, 3), jnp.float32),
        scratch_shapes=[
            pltpu.SMEM((1, 1), jnp.float32),
            pltpu.SMEM((1, 1), jnp.float32),
            pltpu.VMEM((Br, 1), jnp.float32),
            pltpu.VMEM((Br, 1), jnp.float32),
        ],
        compiler_params=pltpu.CompilerParams(
            dimension_semantics=("arbitrary",),
        ),
    )(output, target.reshape(B, 1), oh2)

    vc = (V // 128) * 128
    tw = 128
    res = pl.pallas_call(
        functools.partial(_combine_body, B, Bs, V, vc),
        grid=(1,),
        in_specs=[
            pl.BlockSpec(memory_space=pltpu.SMEM),
            pl.BlockSpec((1, tw), lambda k: (0, 0)),
            pl.BlockSpec((1, Bs), lambda k: (0, 0)),
            pl.BlockSpec((Bs, 1), lambda k: (0, 0)),
            pl.BlockSpec((Bs, 1), lambda k: (0, 0)),
            pl.BlockSpec((1, Bs), lambda k: (0, 0)),
            pl.BlockSpec((Bs, tw), lambda k: (0, vc // tw)),
            pl.BlockSpec((Bs, 1), lambda k: (0, 0)),
        ],
        out_specs=pl.BlockSpec(memory_space=pltpu.SMEM),
        out_shape=jax.ShapeDtypeStruct((1, 1), jnp.float32),
    )(tc_out, oh2, s_arr.reshape(1, Bs), g_arr.reshape(Bs, 1),
      oht_arr.reshape(Bs, 1), i_arr.reshape(1, Bs),
      output, target.reshape(B, 1))
    return res[0, 0]
